# Initial kernel scaffold; baseline (speedup 1.0000x reference)
import jax
import jax.numpy as jnp
from jax.experimental import pallas as pl


def kernel(x, edge_index, W1, b1, W2, b2, W3, b3, Wl, bl):
    raise NotImplementedError("write your pallas kernel here")


# speedup vs baseline: 4.5001x; 4.5001x over previous
import functools

import jax
import jax.numpy as jnp
from jax import lax
from jax.experimental import pallas as pl
from jax.experimental.pallas import tpu as pltpu
from jax.experimental.pallas import tpu_sc as plsc

N = 10000
E = 160000
NP = 10112
NW = 32
CH = 128
CPT = 40
RPT = NP // 16


def _agg_body(src_hbm, dst_hbm, hp_hbm, zz_hbm, out_hbm, sidx, didx, rows, acc):
    c = lax.axis_index("c")
    s = lax.axis_index("s")
    wid = s * 2 + c
    pltpu.sync_copy(zz_hbm, acc.at[pl.ds(s * RPT, RPT)])
    plsc.subcore_barrier()

    def chunk(j, carry):
        pltpu.sync_copy(src_hbm.at[wid, j], sidx.at[0])
        pltpu.sync_copy(dst_hbm.at[wid, j], didx.at[0])
        pltpu.sync_copy(hp_hbm.at[sidx.at[0]], rows)
        pltpu.sync_copy(rows, acc.at[didx.at[0]], add=True)
        return carry

    lax.fori_loop(0, CPT, chunk, 0)
    plsc.subcore_barrier()
    pltpu.sync_copy(acc.at[pl.ds(s * RPT, RPT)],
                    out_hbm.at[c, pl.ds(s * RPT, RPT)])


@functools.cache
def _agg():
    mesh = plsc.VectorSubcoreMesh(core_axis_name="c", subcore_axis_name="s")
    return functools.partial(
        pl.kernel,
        out_type=jax.ShapeDtypeStruct((2, NP, 128), jnp.float32),
        mesh=mesh,
        scratch_types=[
            pltpu.VMEM((1, CH), jnp.int32),
            pltpu.VMEM((1, CH), jnp.int32),
            pltpu.VMEM((CH, 128), jnp.float32),
            pltpu.VMEM_SHARED((NP, 128), jnp.float32),
        ],
    )(_agg_body)


_B = 1000


def _row_spec(w):
    return pl.BlockSpec((_B, w), lambda i: (i, 0))


def _pair_spec(w):
    return pl.BlockSpec((2, _B, w), lambda i: (0, i, 0))


def _full_spec(r, cdim):
    return pl.BlockSpec((r, cdim), lambda i: (0, 0))


def _f0_body(degp, x, dinv8, xp0, xp1):
    deg = degp[0] + degp[1] + 1.0
    d8 = lax.rsqrt(deg)
    dinv8[...] = d8
    d1 = d8[:, :1]
    xp0[...] = x[:, :128] * d1
    xp1[...] = x[:, 128:] * d1


def _f0(degp, x):
    return pl.pallas_call(
        _f0_body,
        grid=(N // _B,),
        in_specs=[_pair_spec(8), _row_spec(256)],
        out_specs=[_row_spec(8), _row_spec(128), _row_spec(128)],
        out_shape=[
            jax.ShapeDtypeStruct((N, 8), jnp.float32),
            jax.ShapeDtypeStruct((N, 128), jnp.float32),
            jax.ShapeDtypeStruct((N, 128), jnp.float32),
        ],
    )(degp, x)


def _f1_body(p0, p1, xp0, xp1, dinv8, w1, b1, w2, o0, o1):
    d1 = dinv8[:, :1]
    a0 = (p0[0] + p0[1] + xp0[...]) * d1
    a1 = (p1[0] + p1[1] + xp1[...]) * d1
    agg = jnp.concatenate([a0, a1], axis=1)
    h1 = jnp.dot(agg, w1[...], preferred_element_type=jnp.float32)
    h1 = jnp.maximum(h1 + b1[0], 0.0)
    t2 = jnp.dot(h1, w2[...], preferred_element_type=jnp.float32) * d1
    o0[...] = t2[:, :128]
    o1[...] = t2[:, 128:]


def _f1(p0, p1, xp0, xp1, dinv8, w1, b1, w2):
    return pl.pallas_call(
        _f1_body,
        grid=(N // _B,),
        in_specs=[_pair_spec(128), _pair_spec(128), _row_spec(128),
                  _row_spec(128), _row_spec(8), _full_spec(256, 512),
                  _full_spec(1, 512), _full_spec(512, 256)],
        out_specs=[_row_spec(128), _row_spec(128)],
        out_shape=[jax.ShapeDtypeStruct((N, 128), jnp.float32),
                   jax.ShapeDtypeStruct((N, 128), jnp.float32)],
    )(p0, p1, xp0, xp1, dinv8, w1, b1, w2)


def _f2_body(p0, p1, t0, t1, dinv8, b2, w3, o):
    d1 = dinv8[:, :1]
    a0 = (p0[0] + p0[1] + t0[...]) * d1
    a1 = (p1[0] + p1[1] + t1[...]) * d1
    agg = jnp.concatenate([a0, a1], axis=1) + b2[0]
    h2 = jnp.maximum(agg, 0.0)
    o[...] = jnp.dot(h2, w3[...], preferred_element_type=jnp.float32) * d1


def _f2(p0, p1, t0, t1, dinv8, b2, w3):
    return pl.pallas_call(
        _f2_body,
        grid=(N // _B,),
        in_specs=[_pair_spec(128), _pair_spec(128), _row_spec(128),
                  _row_spec(128), _row_spec(8), _full_spec(1, 256),
                  _full_spec(256, 128)],
        out_specs=_row_spec(128),
        out_shape=jax.ShapeDtypeStruct((N, 128), jnp.float32),
    )(p0, p1, t0, t1, dinv8, b2, w3)


def _f3_body(p, t, dinv8, b3, wl, bl, o):
    d1 = dinv8[:, :1]
    agg = (p[0] + p[1] + t[...]) * d1 + b3[0]
    h3 = jnp.maximum(agg, 0.0)
    o[...] = jnp.dot(h3, wl[...], preferred_element_type=jnp.float32) + bl[0]


def _f3(p, t, dinv8, b3, wl, bl):
    return pl.pallas_call(
        _f3_body,
        grid=(N // _B,),
        in_specs=[_pair_spec(128), _row_spec(128), _row_spec(8),
                  _full_spec(1, 128), _full_spec(128, 128),
                  _full_spec(1, 128)],
        out_specs=_row_spec(128),
        out_shape=jax.ShapeDtypeStruct((N, 128), jnp.float32),
    )(p, t, dinv8, b3, wl, bl)


def kernel(x, edge_index, W1, b1, W2, b2, W3, b3, Wl, bl):
    src = edge_index[0].astype(jnp.int32)
    dst = edge_index[1].astype(jnp.int32)
    pad = NW * CPT * CH - E
    src3 = jnp.concatenate([src, jnp.zeros((pad,), jnp.int32)]).reshape(NW, CPT, CH)
    dst3 = jnp.concatenate([dst, jnp.full((pad,), N, jnp.int32)]).reshape(NW, CPT, CH)
    zz = jnp.zeros((RPT, 128), jnp.float32)
    ones_n = jnp.ones((N, 128), jnp.float32)

    agg = _agg()
    degp = agg(src3, dst3, ones_n, zz)[:, :N, :8]
    dinv8, xp0, xp1 = _f0(degp, x)
    p0 = agg(src3, dst3, xp0, zz)[:, :N, :]
    p1 = agg(src3, dst3, xp1, zz)[:, :N, :]
    t0, t1 = _f1(p0, p1, xp0, xp1, dinv8, W1, b1.reshape(1, -1), W2)

    q0 = agg(src3, dst3, t0, zz)[:, :N, :]
    q1 = agg(src3, dst3, t1, zz)[:, :N, :]
    u = _f2(q0, q1, t0, t1, dinv8, b2.reshape(1, -1), W3)

    r = agg(src3, dst3, u, zz)[:, :N, :]
    wlp = jnp.pad(Wl, ((0, 0), (0, 127)))
    blp = jnp.pad(bl, (0, 127)).reshape(1, -1)
    out = _f3(r, u, dinv8, b3.reshape(1, -1), wlp, blp)
    return out[:, :1]


# speedup vs baseline: 4.9791x; 1.1065x over previous
import functools

import jax
import jax.numpy as jnp
from jax import lax
from jax.experimental import pallas as pl
from jax.experimental.pallas import tpu as pltpu
from jax.experimental.pallas import tpu_sc as plsc

N = 10000
E = 160000
NP = 10112
NW = 32
CH = 64
CPT = 80
RPT = NP // 16


NBUF = 3


def _agg_body(src_hbm, dst_hbm, hp_hbm, zz_hbm, out_hbm, sidx, didx, rows, acc,
              *sems):
    gsem = sems[:NBUF]
    ssem = sems[NBUF:]
    c = lax.axis_index("c")
    s = lax.axis_index("s")
    wid = s * 2 + c
    pltpu.sync_copy(src_hbm.at[wid], sidx)
    pltpu.sync_copy(dst_hbm.at[wid], didx)
    pltpu.sync_copy(zz_hbm, acc.at[pl.ds(s * RPT, RPT)])
    plsc.subcore_barrier()

    gd = [None] * NBUF
    sd = [None] * NBUF
    GLAG = 1
    for j in range(CPT + GLAG):
        if j < CPT:
            b = j % NBUF
            if j >= NBUF:
                sd[b].wait()
            gd[b] = pltpu.async_copy(hp_hbm.at[sidx.at[j]], rows.at[b],
                                     gsem[b])
        if j >= GLAG:
            jj = j - GLAG
            bb = jj % NBUF
            gd[bb].wait()
            sd[bb] = pltpu.async_copy(rows.at[bb], acc.at[didx.at[jj]],
                                      ssem[bb], add=True)
    for b in range(NBUF):
        sd[b].wait()
    plsc.subcore_barrier()
    pltpu.sync_copy(acc.at[pl.ds(s * RPT, RPT)],
                    out_hbm.at[c, pl.ds(s * RPT, RPT)])


@functools.cache
def _agg():
    mesh = plsc.VectorSubcoreMesh(core_axis_name="c", subcore_axis_name="s")
    return functools.partial(
        pl.kernel,
        out_type=jax.ShapeDtypeStruct((2, NP, 128), jnp.float32),
        mesh=mesh,
        scratch_types=[
            pltpu.VMEM((CPT, CH), jnp.int32),
            pltpu.VMEM((CPT, CH), jnp.int32),
            pltpu.VMEM((NBUF, CH, 128), jnp.float32),
            pltpu.VMEM_SHARED((NP, 128), jnp.float32),
        ] + [pltpu.SemaphoreType.DMA] * (2 * NBUF),
    )(_agg_body)


_B = 1000


def _row_spec(w):
    return pl.BlockSpec((_B, w), lambda i: (i, 0))


def _pair_spec(w):
    return pl.BlockSpec((2, _B, w), lambda i: (0, i, 0))


def _full_spec(r, cdim):
    return pl.BlockSpec((r, cdim), lambda i: (0, 0))


def _f0_body(degp, x, dinv8, xp0, xp1):
    deg = degp[0] + degp[1] + 1.0
    d8 = lax.rsqrt(deg)
    dinv8[...] = d8
    d1 = d8[:, :1]
    xp0[...] = x[:, :128] * d1
    xp1[...] = x[:, 128:] * d1


def _f0(degp, x):
    return pl.pallas_call(
        _f0_body,
        grid=(N // _B,),
        in_specs=[_pair_spec(8), _row_spec(256)],
        out_specs=[_row_spec(8), _row_spec(128), _row_spec(128)],
        out_shape=[
            jax.ShapeDtypeStruct((N, 8), jnp.float32),
            jax.ShapeDtypeStruct((N, 128), jnp.float32),
            jax.ShapeDtypeStruct((N, 128), jnp.float32),
        ],
    )(degp, x)


def _f1_body(p0, p1, xp0, xp1, dinv8, w1, b1, w2, o0, o1):
    d1 = dinv8[:, :1]
    a0 = (p0[0] + p0[1] + xp0[...]) * d1
    a1 = (p1[0] + p1[1] + xp1[...]) * d1
    agg = jnp.concatenate([a0, a1], axis=1)
    h1 = jnp.dot(agg, w1[...], preferred_element_type=jnp.float32)
    h1 = jnp.maximum(h1 + b1[0], 0.0)
    t2 = jnp.dot(h1, w2[...], preferred_element_type=jnp.float32) * d1
    o0[...] = t2[:, :128]
    o1[...] = t2[:, 128:]


def _f1(p0, p1, xp0, xp1, dinv8, w1, b1, w2):
    return pl.pallas_call(
        _f1_body,
        grid=(N // _B,),
        in_specs=[_pair_spec(128), _pair_spec(128), _row_spec(128),
                  _row_spec(128), _row_spec(8), _full_spec(256, 512),
                  _full_spec(1, 512), _full_spec(512, 256)],
        out_specs=[_row_spec(128), _row_spec(128)],
        out_shape=[jax.ShapeDtypeStruct((N, 128), jnp.float32),
                   jax.ShapeDtypeStruct((N, 128), jnp.float32)],
    )(p0, p1, xp0, xp1, dinv8, w1, b1, w2)


def _f2_body(p0, p1, t0, t1, dinv8, b2, w3, o):
    d1 = dinv8[:, :1]
    a0 = (p0[0] + p0[1] + t0[...]) * d1
    a1 = (p1[0] + p1[1] + t1[...]) * d1
    agg = jnp.concatenate([a0, a1], axis=1) + b2[0]
    h2 = jnp.maximum(agg, 0.0)
    o[...] = jnp.dot(h2, w3[...], preferred_element_type=jnp.float32) * d1


def _f2(p0, p1, t0, t1, dinv8, b2, w3):
    return pl.pallas_call(
        _f2_body,
        grid=(N // _B,),
        in_specs=[_pair_spec(128), _pair_spec(128), _row_spec(128),
                  _row_spec(128), _row_spec(8), _full_spec(1, 256),
                  _full_spec(256, 128)],
        out_specs=_row_spec(128),
        out_shape=jax.ShapeDtypeStruct((N, 128), jnp.float32),
    )(p0, p1, t0, t1, dinv8, b2, w3)


def _f3_body(p, t, dinv8, b3, wl, bl, o):
    d1 = dinv8[:, :1]
    agg = (p[0] + p[1] + t[...]) * d1 + b3[0]
    h3 = jnp.maximum(agg, 0.0)
    o[...] = jnp.dot(h3, wl[...], preferred_element_type=jnp.float32) + bl[0]


def _f3(p, t, dinv8, b3, wl, bl):
    return pl.pallas_call(
        _f3_body,
        grid=(N // _B,),
        in_specs=[_pair_spec(128), _row_spec(128), _row_spec(8),
                  _full_spec(1, 128), _full_spec(128, 128),
                  _full_spec(1, 128)],
        out_specs=_row_spec(128),
        out_shape=jax.ShapeDtypeStruct((N, 128), jnp.float32),
    )(p, t, dinv8, b3, wl, bl)


def kernel(x, edge_index, W1, b1, W2, b2, W3, b3, Wl, bl):
    src = edge_index[0].astype(jnp.int32)
    dst = edge_index[1].astype(jnp.int32)
    pad = NW * CPT * CH - E
    src3 = jnp.concatenate([src, jnp.zeros((pad,), jnp.int32)]).reshape(NW, CPT, CH)
    dst3 = jnp.concatenate([dst, jnp.full((pad,), N, jnp.int32)]).reshape(NW, CPT, CH)
    zz = jnp.zeros((RPT, 128), jnp.float32)
    ones_n = jnp.ones((N, 128), jnp.float32)

    agg = _agg()
    degp = agg(src3, dst3, ones_n, zz)[:, :N, :8]
    dinv8, xp0, xp1 = _f0(degp, x)
    p0 = agg(src3, dst3, xp0, zz)[:, :N, :]
    p1 = agg(src3, dst3, xp1, zz)[:, :N, :]
    t0, t1 = _f1(p0, p1, xp0, xp1, dinv8, W1, b1.reshape(1, -1), W2)

    q0 = agg(src3, dst3, t0, zz)[:, :N, :]
    q1 = agg(src3, dst3, t1, zz)[:, :N, :]
    u = _f2(q0, q1, t0, t1, dinv8, b2.reshape(1, -1), W3)

    r = agg(src3, dst3, u, zz)[:, :N, :]
    wlp = jnp.pad(Wl, ((0, 0), (0, 127)))
    blp = jnp.pad(bl, (0, 127)).reshape(1, -1)
    out = _f3(r, u, dinv8, b3.reshape(1, -1), wlp, blp)
    return out[:, :1]


# speedup vs baseline: 5.4642x; 1.0974x over previous
import functools

import jax
import jax.numpy as jnp
from jax import lax
from jax.experimental import pallas as pl
from jax.experimental.pallas import tpu as pltpu
from jax.experimental.pallas import tpu_sc as plsc

N = 10000
E = 160000
NP = 10112
NW = 32
CH = 64
CPT = 80
RPT = NP // 16
NBUF = 3
GLAG = 1


def _agg_body(nslab, src_hbm, dst_hbm, hp_hbm, zz_hbm, out_hbm,
              sidx, didx, rows, acc, *sems):
    gsem = sems[:NBUF]
    ssem = sems[NBUF:]
    c = lax.axis_index("c")
    s = lax.axis_index("s")
    wid = s * 2 + c
    pltpu.sync_copy(src_hbm.at[wid], sidx)
    pltpu.sync_copy(dst_hbm.at[wid], didx)

    for slab in range(nslab):
        pltpu.sync_copy(zz_hbm, acc.at[pl.ds(s * RPT, RPT)])
        plsc.subcore_barrier()

        gd = [None] * NBUF
        sd = [None] * NBUF
        hp = hp_hbm.at[slab]
        for j in range(CPT + GLAG):
            if j < CPT:
                b = j % NBUF
                if j >= NBUF:
                    sd[b].wait()
                gd[b] = pltpu.async_copy(hp.at[sidx.at[j]], rows.at[b],
                                         gsem[b])
            if j >= GLAG:
                jj = j - GLAG
                bb = jj % NBUF
                gd[bb].wait()
                sd[bb] = pltpu.async_copy(rows.at[bb], acc.at[didx.at[jj]],
                                          ssem[bb], add=True)
        for b in range(NBUF):
            sd[b].wait()
        plsc.subcore_barrier()
        pltpu.sync_copy(acc.at[pl.ds(s * RPT, RPT)],
                        out_hbm.at[slab, c, pl.ds(s * RPT, RPT)])


@functools.cache
def _agg(nslab):
    mesh = plsc.VectorSubcoreMesh(core_axis_name="c", subcore_axis_name="s")
    return functools.partial(
        pl.kernel,
        out_type=jax.ShapeDtypeStruct((nslab, 2, NP, 128), jnp.float32),
        mesh=mesh,
        scratch_types=[
            pltpu.VMEM((CPT, CH), jnp.int32),
            pltpu.VMEM((CPT, CH), jnp.int32),
            pltpu.VMEM((NBUF, CH, 128), jnp.float32),
            pltpu.VMEM_SHARED((NP, 128), jnp.float32),
        ] + [pltpu.SemaphoreType.DMA] * (2 * NBUF),
    )(functools.partial(_agg_body, nslab))


_B = 1000


def _row_spec(w):
    return pl.BlockSpec((_B, w), lambda i: (i, 0))


def _slab_spec(nslab, w):
    return pl.BlockSpec((nslab, _B, w), lambda i: (0, i, 0))


def _part_spec(nslab, w):
    return pl.BlockSpec((nslab, 2, _B, w), lambda i: (0, 0, i, 0))


def _full_spec(r, cdim):
    return pl.BlockSpec((r, cdim), lambda i: (0, 0))


def _f0_body(degp, x, dinv8, xp):
    deg = degp[0] + degp[1] + 1.0
    d8 = lax.rsqrt(deg)
    dinv8[...] = d8
    d1 = d8[:, :1]
    xp[0] = x[:, :128] * d1
    xp[1] = x[:, 128:] * d1


def _f0(degp, x):
    return pl.pallas_call(
        _f0_body,
        grid=(N // _B,),
        in_specs=[pl.BlockSpec((2, _B, 8), lambda i: (0, i, 0)),
                  _row_spec(256)],
        out_specs=[_row_spec(8), _slab_spec(2, 128)],
        out_shape=[
            jax.ShapeDtypeStruct((N, 8), jnp.float32),
            jax.ShapeDtypeStruct((2, N, 128), jnp.float32),
        ],
    )(degp, x)


def _f1_body(p, xp, dinv8, w1, b1, w2, t):
    d1 = dinv8[:, :1]
    a0 = (p[0, 0] + p[0, 1] + xp[0]) * d1
    a1 = (p[1, 0] + p[1, 1] + xp[1]) * d1
    agg = jnp.concatenate([a0, a1], axis=1)
    h1 = jnp.dot(agg, w1[...], preferred_element_type=jnp.float32)
    h1 = jnp.maximum(h1 + b1[0], 0.0)
    t2 = jnp.dot(h1, w2[...], preferred_element_type=jnp.float32) * d1
    t[0] = t2[:, :128]
    t[1] = t2[:, 128:]


def _f1(p, xp, dinv8, w1, b1, w2):
    return pl.pallas_call(
        _f1_body,
        grid=(N // _B,),
        in_specs=[_part_spec(2, 128), _slab_spec(2, 128), _row_spec(8),
                  _full_spec(256, 512), _full_spec(1, 512),
                  _full_spec(512, 256)],
        out_specs=_slab_spec(2, 128),
        out_shape=jax.ShapeDtypeStruct((2, N, 128), jnp.float32),
    )(p, xp, dinv8, w1, b1, w2)


def _f2_body(q, t, dinv8, b2, w3, o):
    d1 = dinv8[:, :1]
    a0 = (q[0, 0] + q[0, 1] + t[0]) * d1
    a1 = (q[1, 0] + q[1, 1] + t[1]) * d1
    agg = jnp.concatenate([a0, a1], axis=1) + b2[0]
    h2 = jnp.maximum(agg, 0.0)
    o[...] = jnp.dot(h2, w3[...], preferred_element_type=jnp.float32) * d1


def _f2(q, t, dinv8, b2, w3):
    return pl.pallas_call(
        _f2_body,
        grid=(N // _B,),
        in_specs=[_part_spec(2, 128), _slab_spec(2, 128), _row_spec(8),
                  _full_spec(1, 256), _full_spec(256, 128)],
        out_specs=_row_spec(128),
        out_shape=jax.ShapeDtypeStruct((N, 128), jnp.float32),
    )(q, t, dinv8, b2, w3)


def _f3_body(r, t, dinv8, b3, wl, bl, o):
    d1 = dinv8[:, :1]
    agg = (r[0] + r[1] + t[...]) * d1 + b3[0]
    h3 = jnp.maximum(agg, 0.0)
    o[...] = jnp.dot(h3, wl[...], preferred_element_type=jnp.float32) + bl[0]


def _f3(r, t, dinv8, b3, wl, bl):
    return pl.pallas_call(
        _f3_body,
        grid=(N // _B,),
        in_specs=[pl.BlockSpec((2, _B, 128), lambda i: (0, i, 0)),
                  _row_spec(128), _row_spec(8), _full_spec(1, 128),
                  _full_spec(128, 128), _full_spec(1, 128)],
        out_specs=_row_spec(128),
        out_shape=jax.ShapeDtypeStruct((N, 128), jnp.float32),
    )(r, t, dinv8, b3, wl, bl)


def kernel(x, edge_index, W1, b1, W2, b2, W3, b3, Wl, bl):
    src = edge_index[0].astype(jnp.int32)
    dst = edge_index[1].astype(jnp.int32)
    pad = NW * CPT * CH - E
    src3 = jnp.concatenate([src, jnp.zeros((pad,), jnp.int32)]).reshape(NW, CPT, CH)
    dst3 = jnp.concatenate([dst, jnp.full((pad,), N, jnp.int32)]).reshape(NW, CPT, CH)
    zz = jnp.zeros((RPT, 128), jnp.float32)
    ones_n = jnp.ones((1, N, 128), jnp.float32)

    degp = _agg(1)(src3, dst3, ones_n, zz)[0, :, :N, :8]
    dinv8, xp = _f0(degp, x)

    p = _agg(2)(src3, dst3, xp, zz)[:, :, :N, :]
    t = _f1(p, xp, dinv8, W1, b1.reshape(1, -1), W2)

    q = _agg(2)(src3, dst3, t, zz)[:, :, :N, :]
    u = _f2(q, t, dinv8, b2.reshape(1, -1), W3)

    r = _agg(1)(src3, dst3, u.reshape(1, N, 128), zz)[0, :, :N, :]
    wlp = jnp.pad(Wl, ((0, 0), (0, 127)))
    blp = jnp.pad(bl, (0, 127)).reshape(1, -1)
    out = _f3(r, u, dinv8, b3.reshape(1, -1), wlp, blp)
    return out[:, :1]
